# baseline (speedup 1.0000x reference)
import functools

import jax
import jax.numpy as jnp
from jax import lax
from jax.experimental import pallas as pl
from jax.experimental.pallas import tpu as pltpu
from jax.experimental.pallas import tpu_sc as plsc

N = 4096
D = 1024
H = 2048
E = 8
K = 2
BLK = 256
G_MAX = N * K // BLK + E
R_MAX = G_MAX * BLK
TBLK = 256
NTB = N // TBLK
NW = 32
TPW = N // NW
CH = 32
NCH = TPW // CH

_F32 = jnp.float32
_I32 = jnp.int32


def _gelu(v):
    return 0.5 * v * (1.0 + lax.erf(v * 0.7071067811865476))


def _plan_kernel(x_ref, wr_ref, br_ref, g0_ref, g1_ref, dest_ref, be_ref,
                 e0_scr, e1_scr):
    i = pl.program_id(0)
    logits = jnp.dot(x_ref[...], wr_ref[...],
                     preferred_element_type=_F32) + br_ref[...]
    tr = logits.T
    io = lax.broadcasted_iota(_I32, (E, TBLK), 0)
    m0 = jnp.max(tr, axis=0, keepdims=True)
    i0 = jnp.min(jnp.where(tr == m0, io, E), axis=0, keepdims=True)
    l2 = jnp.where(io == i0, _F32(-1e30), tr)
    m1 = jnp.max(l2, axis=0, keepdims=True)
    i1 = jnp.min(jnp.where(l2 == m1, io, E), axis=0, keepdims=True)
    e1m = jnp.exp(m1 - m0)
    s = 1.0 + e1m
    g0_ref[...] = (1.0 / s).reshape(1, 1, TBLK)
    g1_ref[...] = (e1m / s).reshape(1, 1, TBLK)
    e0_scr[pl.ds(i, 1), :] = i0
    e1_scr[pl.ds(i, 1), :] = i1

    @pl.when(i == NTB - 1)
    def _():
        e0 = e0_scr[...]
        e1 = e1_scr[...]
        rio = lax.broadcasted_iota(_I32, (TBLK, TBLK), 0)
        cio = lax.broadcasted_iota(_I32, (TBLK, TBLK), 1)
        up = (rio < cio).astype(_F32)
        r2 = lax.broadcasted_iota(_I32, (2 * NTB, 2 * NTB), 0)
        c2 = lax.broadcasted_iota(_I32, (2 * NTB, 2 * NTB), 1)
        lo = (c2 < r2).astype(_F32)
        bi = (lax.broadcasted_iota(_I32, (1, TBLK), 1) * BLK).astype(_F32)
        dest = jnp.zeros((2 * NTB, TBLK), _F32)
        bev = jnp.zeros((1, TBLK), _I32)
        start = _F32(0.0)
        for e in range(E):
            oh = jnp.concatenate([(e0 == e), (e1 == e)],
                                 axis=0).astype(_F32)
            rs = jnp.sum(oh, axis=1, keepdims=True)
            excl_rows = jnp.dot(lo, rs, preferred_element_type=_F32)
            excl_in = jnp.dot(oh, up, preferred_element_type=_F32)
            cnt = jnp.sum(rs)
            padded = jnp.ceil(cnt / BLK) * BLK
            dest = jnp.where(oh > 0, start + excl_rows + excl_in, dest)
            start = start + padded
            bev = bev + jnp.where(bi >= start, 1, 0).astype(_I32)
        dest_ref[...] = dest.astype(_I32)
        be_ref[...] = jnp.minimum(bev, E - 1)


def _plan_call(xf, wr, br2):
    return pl.pallas_call(
        _plan_kernel,
        grid=(NTB,),
        in_specs=[
            pl.BlockSpec((TBLK, D), lambda i: (i, 0)),
            pl.BlockSpec((D, E), lambda i: (0, 0)),
            pl.BlockSpec((1, E), lambda i: (0, 0)),
        ],
        out_specs=[
            pl.BlockSpec((1, 1, TBLK), lambda i: (i, 0, 0)),
            pl.BlockSpec((1, 1, TBLK), lambda i: (i, 0, 0)),
            pl.BlockSpec((2 * NTB, TBLK), lambda i: (0, 0)),
            pl.BlockSpec((1, TBLK), lambda i: (0, 0)),
        ],
        out_shape=[
            jax.ShapeDtypeStruct((NTB, 1, TBLK), _F32),
            jax.ShapeDtypeStruct((NTB, 1, TBLK), _F32),
            jax.ShapeDtypeStruct((2 * NTB, TBLK), _I32),
            jax.ShapeDtypeStruct((1, TBLK), _I32),
        ],
        scratch_shapes=[
            pltpu.VMEM((NTB, TBLK), _I32),
            pltpu.VMEM((NTB, TBLK), _I32),
        ],
    )(xf, wr, br2)


@functools.cache
def _sc_dispatch_kernel():
    mesh = plsc.VectorSubcoreMesh(core_axis_name="c", subcore_axis_name="s")

    @functools.partial(
        pl.kernel,
        out_type=jax.ShapeDtypeStruct((R_MAX, D), _F32),
        mesh=mesh,
        scratch_types=[
            pltpu.VMEM((CH, D), _F32),
            pltpu.VMEM((CH,), _I32),
            pltpu.VMEM((CH,), _I32),
            pltpu.SemaphoreType.DMA,
            pltpu.SemaphoreType.DMA,
        ],
    )
    def dispatch(x_hbm, dplan_hbm, xg_hbm, rows_v, idx0_v, idx1_v, sem0, sem1):
        wid = lax.axis_index("s") * 2 + lax.axis_index("c")
        for c in range(NCH):
            base = wid * TPW + c * CH
            pltpu.sync_copy(x_hbm.at[pl.ds(base, CH)], rows_v)
            pltpu.sync_copy(dplan_hbm.at[wid, c, 0], idx0_v)
            pltpu.sync_copy(dplan_hbm.at[wid, c, 1], idx1_v)
            cp0 = pltpu.async_copy(rows_v, xg_hbm.at[idx0_v], sem0)
            cp1 = pltpu.async_copy(rows_v, xg_hbm.at[idx1_v], sem1)
            cp0.wait()
            cp1.wait()

    return dispatch


def _sc_dispatch(xf, dplan):
    return _sc_dispatch_kernel()(xf, dplan)


def _expert_kernel(be_ref, xg_ref, w1_ref, b1_ref, w2_ref, b2_ref, y_ref):
    h = _gelu(jnp.dot(xg_ref[...].astype(jnp.bfloat16), w1_ref[0],
                      preferred_element_type=_F32) + b1_ref[0])
    y_ref[...] = jnp.dot(h.astype(jnp.bfloat16), w2_ref[0],
                         preferred_element_type=_F32) + b2_ref[0]


def _expert_call(be, xg, ew1, eb1, ew2, eb2):
    grid_spec = pltpu.PrefetchScalarGridSpec(
        num_scalar_prefetch=1,
        grid=(G_MAX,),
        in_specs=[
            pl.BlockSpec((BLK, D), lambda i, be: (i, 0)),
            pl.BlockSpec((1, D, H), lambda i, be: (be[i], 0, 0)),
            pl.BlockSpec((1, 1, H), lambda i, be: (be[i], 0, 0)),
            pl.BlockSpec((1, H, D), lambda i, be: (be[i], 0, 0)),
            pl.BlockSpec((1, 1, D), lambda i, be: (be[i], 0, 0)),
        ],
        out_specs=pl.BlockSpec((BLK, D), lambda i, be: (i, 0)),
    )
    return pl.pallas_call(
        _expert_kernel,
        grid_spec=grid_spec,
        out_shape=jax.ShapeDtypeStruct((R_MAX, D), _F32),
    )(be, xg, ew1, eb1.reshape(E, 1, H), ew2, eb2.reshape(E, 1, D))


@functools.cache
def _sc_collect_kernel():
    mesh = plsc.VectorSubcoreMesh(core_axis_name="c", subcore_axis_name="s")

    @functools.partial(
        pl.kernel,
        out_type=(
            jax.ShapeDtypeStruct((N, D), _F32),
            jax.ShapeDtypeStruct((N, D), _F32),
        ),
        mesh=mesh,
        scratch_types=[
            pltpu.VMEM((CH, D), _F32),
            pltpu.VMEM((CH, D), _F32),
            pltpu.VMEM((CH,), _I32),
            pltpu.VMEM((CH,), _I32),
            pltpu.SemaphoreType.DMA,
            pltpu.SemaphoreType.DMA,
        ],
    )
    def collect(y_hbm, dplan_hbm, yg0_hbm, yg1_hbm, r0_v, r1_v,
                idx0_v, idx1_v, sem0, sem1):
        wid = lax.axis_index("s") * 2 + lax.axis_index("c")
        for c in range(NCH):
            base = wid * TPW + c * CH
            pltpu.sync_copy(dplan_hbm.at[wid, c, 0], idx0_v)
            pltpu.sync_copy(dplan_hbm.at[wid, c, 1], idx1_v)
            cp0 = pltpu.async_copy(y_hbm.at[idx0_v], r0_v, sem0)
            cp1 = pltpu.async_copy(y_hbm.at[idx1_v], r1_v, sem1)
            cp0.wait()
            cp1.wait()
            pltpu.sync_copy(r0_v, yg0_hbm.at[pl.ds(base, CH)])
            pltpu.sync_copy(r1_v, yg1_hbm.at[pl.ds(base, CH)])

    return collect


def _sc_collect(y, dplan):
    return _sc_collect_kernel()(y, dplan)


def _shared_kernel(x_ref, w1_ref, b1_ref, w2_ref, b2_ref,
                   yg0_ref, yg1_ref, g0_ref, g1_ref, o_ref):
    h = _gelu(jnp.dot(x_ref[...].astype(jnp.bfloat16), w1_ref[...],
                      preferred_element_type=_F32) + b1_ref[...])
    sh = jnp.dot(h.astype(jnp.bfloat16), w2_ref[...],
                 preferred_element_type=_F32) + b2_ref[...]
    o_ref[...] = sh + g0_ref[...] * yg0_ref[...] + g1_ref[...] * yg1_ref[...]


def _shared_call(xf, sw1, sb1r, sw2, sb2r, yg0, yg1, g0c, g1c):
    return pl.pallas_call(
        _shared_kernel,
        grid=(NTB,),
        in_specs=[
            pl.BlockSpec((TBLK, D), lambda i: (i, 0)),
            pl.BlockSpec((D, H), lambda i: (0, 0)),
            pl.BlockSpec((1, H), lambda i: (0, 0)),
            pl.BlockSpec((H, D), lambda i: (0, 0)),
            pl.BlockSpec((1, D), lambda i: (0, 0)),
            pl.BlockSpec((TBLK, D), lambda i: (i, 0)),
            pl.BlockSpec((TBLK, D), lambda i: (i, 0)),
            pl.BlockSpec((TBLK, 1), lambda i: (i, 0)),
            pl.BlockSpec((TBLK, 1), lambda i: (i, 0)),
        ],
        out_specs=pl.BlockSpec((TBLK, D), lambda i: (i, 0)),
        out_shape=jax.ShapeDtypeStruct((N, D), _F32),
    )(xf, sw1, sb1r, sw2, sb2r, yg0, yg1, g0c, g1c)


def kernel(x, Wr, br, sW1, sb1, sW2, sb2, eW1, eb1, eW2, eb2):
    b, s, d = x.shape
    xf = x.reshape(N, D)
    g0, g1, dest2, bev = _plan_call(xf, Wr, br.reshape(1, E))
    d0 = dest2[:NTB].reshape(N)
    d1 = dest2[NTB:].reshape(N)
    dplan = jnp.stack(
        [d0.reshape(NW, NCH, CH), d1.reshape(NW, NCH, CH)], axis=2)
    be = bev[0, :G_MAX]
    xg = _sc_dispatch(xf, dplan)
    y = _expert_call(be, xg, eW1.astype(jnp.bfloat16), eb1,
                     eW2.astype(jnp.bfloat16), eb2)
    yg0, yg1 = _sc_collect(y, dplan)
    out = _shared_call(xf, sW1.astype(jnp.bfloat16), sb1.reshape(1, H),
                       sW2.astype(jnp.bfloat16), sb2.reshape(1, D),
                       yg0, yg1, g0.reshape(N, 1), g1.reshape(N, 1))
    return out.reshape(b, s, d)


# speedup vs baseline: 1.1750x; 1.1750x over previous
import functools

import jax
import jax.numpy as jnp
from jax import lax
from jax.experimental import pallas as pl
from jax.experimental.pallas import tpu as pltpu
from jax.experimental.pallas import tpu_sc as plsc

N = 4096
D = 1024
H = 2048
E = 8
K = 2
BLK = 256
G_MAX = N * K // BLK + E
R_MAX = G_MAX * BLK
TBLK = 256
NTB = N // TBLK
NW = 32
TPW = N // NW
DCH = 32
NDCH = TPW // DCH
TPC = 256
CCH = 32
NCC = TPC // CCH

_F32 = jnp.float32
_I32 = jnp.int32


def _gelu(v):
    return 0.5 * v * (1.0 + lax.erf(v * 0.7071067811865476))


def _plan_kernel(x_ref, wr_ref, br_ref, g0_ref, g1_ref, dest_ref, be_ref,
                 e0_scr, e1_scr):
    i = pl.program_id(0)
    logits = jnp.dot(x_ref[...], wr_ref[...],
                     preferred_element_type=_F32) + br_ref[...]
    tr = logits.T
    io = lax.broadcasted_iota(_I32, (E, TBLK), 0)
    m0 = jnp.max(tr, axis=0, keepdims=True)
    i0 = jnp.min(jnp.where(tr == m0, io, E), axis=0, keepdims=True)
    l2 = jnp.where(io == i0, _F32(-1e30), tr)
    m1 = jnp.max(l2, axis=0, keepdims=True)
    i1 = jnp.min(jnp.where(l2 == m1, io, E), axis=0, keepdims=True)
    e1m = jnp.exp(m1 - m0)
    s = 1.0 + e1m
    g0_ref[...] = (1.0 / s).reshape(1, 1, TBLK)
    g1_ref[...] = (e1m / s).reshape(1, 1, TBLK)
    e0_scr[pl.ds(i, 1), :] = i0
    e1_scr[pl.ds(i, 1), :] = i1

    @pl.when(i == NTB - 1)
    def _():
        e0 = e0_scr[...]
        e1 = e1_scr[...]
        rio = lax.broadcasted_iota(_I32, (TBLK, TBLK), 0)
        cio = lax.broadcasted_iota(_I32, (TBLK, TBLK), 1)
        up = (rio < cio).astype(_F32)
        r2 = lax.broadcasted_iota(_I32, (2 * NTB, 2 * NTB), 0)
        c2 = lax.broadcasted_iota(_I32, (2 * NTB, 2 * NTB), 1)
        lo = (c2 < r2).astype(_F32)
        bi = (lax.broadcasted_iota(_I32, (1, TBLK), 1) * BLK).astype(_F32)
        dest = jnp.zeros((2 * NTB, TBLK), _F32)
        bev = jnp.zeros((1, TBLK), _I32)
        start = _F32(0.0)
        for e in range(E):
            oh = jnp.concatenate([(e0 == e), (e1 == e)],
                                 axis=0).astype(_F32)
            rs = jnp.sum(oh, axis=1, keepdims=True)
            excl_rows = jnp.dot(lo, rs, preferred_element_type=_F32)
            excl_in = jnp.dot(oh, up, preferred_element_type=_F32)
            cnt = jnp.sum(rs)
            padded = jnp.ceil(cnt / BLK) * BLK
            dest = jnp.where(oh > 0, start + excl_rows + excl_in, dest)
            start = start + padded
            bev = bev + jnp.where(bi >= start, 1, 0).astype(_I32)
        dest_ref[...] = dest.astype(_I32)
        be_ref[...] = jnp.minimum(bev, E - 1)


def _plan_call(xf, wr, br2):
    return pl.pallas_call(
        _plan_kernel,
        grid=(NTB,),
        in_specs=[
            pl.BlockSpec((TBLK, D), lambda i: (i, 0)),
            pl.BlockSpec((D, E), lambda i: (0, 0)),
            pl.BlockSpec((1, E), lambda i: (0, 0)),
        ],
        out_specs=[
            pl.BlockSpec((1, 1, TBLK), lambda i: (i, 0, 0)),
            pl.BlockSpec((1, 1, TBLK), lambda i: (i, 0, 0)),
            pl.BlockSpec((2 * NTB, TBLK), lambda i: (0, 0)),
            pl.BlockSpec((1, TBLK), lambda i: (0, 0)),
        ],
        out_shape=[
            jax.ShapeDtypeStruct((NTB, 1, TBLK), _F32),
            jax.ShapeDtypeStruct((NTB, 1, TBLK), _F32),
            jax.ShapeDtypeStruct((2 * NTB, TBLK), _I32),
            jax.ShapeDtypeStruct((1, TBLK), _I32),
        ],
        scratch_shapes=[
            pltpu.VMEM((NTB, TBLK), _I32),
            pltpu.VMEM((NTB, TBLK), _I32),
        ],
    )(xf, wr, br2)


@functools.cache
def _sc_dispatch_kernel():
    mesh = plsc.VectorSubcoreMesh(core_axis_name="c", subcore_axis_name="s")

    @functools.partial(
        pl.kernel,
        out_type=jax.ShapeDtypeStruct((R_MAX, D), _F32),
        mesh=mesh,
        scratch_types=[
            pltpu.VMEM((DCH, D), _F32),
            pltpu.VMEM((DCH, D), _F32),
            pltpu.VMEM((NDCH, 2, DCH), _I32),
            pltpu.SemaphoreType.DMA,
            pltpu.SemaphoreType.DMA,
        ],
    )
    def dispatch(x_hbm, dest_hbm, xg_hbm, rows0_v, rows1_v, idx_v, sem0, sem1):
        wid = lax.axis_index("s") * 2 + lax.axis_index("c")
        row = wid // 2
        col = (wid % 2) * TPW
        bufs = (rows0_v, rows1_v)
        sems = (sem0, sem1)
        cps = [None] * NDCH
        for c in range(NDCH):
            b = c % 2
            if c >= 2:
                cps[c - 2][0].wait()
                cps[c - 2][1].wait()
            base = wid * TPW + c * DCH
            pltpu.sync_copy(x_hbm.at[pl.ds(base, DCH)], bufs[b])
            pltpu.sync_copy(dest_hbm.at[row, pl.ds(col + c * DCH, DCH)],
                            idx_v.at[c, 0])
            pltpu.sync_copy(dest_hbm.at[NTB + row, pl.ds(col + c * DCH, DCH)],
                            idx_v.at[c, 1])
            cps[c] = (
                pltpu.async_copy(bufs[b], xg_hbm.at[idx_v.at[c, 0]], sems[b]),
                pltpu.async_copy(bufs[b], xg_hbm.at[idx_v.at[c, 1]], sems[b]),
            )
        for c in (NDCH - 2, NDCH - 1):
            cps[c][0].wait()
            cps[c][1].wait()

    return dispatch


def _sc_dispatch(xf, dest2):
    return _sc_dispatch_kernel()(xf, dest2)


def _expert_kernel(be_ref, xg_ref, w1_ref, b1_ref, w2_ref, b2_ref, y_ref):
    h = _gelu(jnp.dot(xg_ref[...], w1_ref[0],
                      preferred_element_type=_F32) + b1_ref[0])
    y_ref[...] = jnp.dot(h, w2_ref[0],
                         preferred_element_type=_F32) + b2_ref[0]


def _expert_call(be, xg, ew1, eb1, ew2, eb2):
    grid_spec = pltpu.PrefetchScalarGridSpec(
        num_scalar_prefetch=1,
        grid=(G_MAX,),
        in_specs=[
            pl.BlockSpec((BLK, D), lambda i, be: (i, 0)),
            pl.BlockSpec((1, D, H), lambda i, be: (be[i], 0, 0)),
            pl.BlockSpec((1, 1, H), lambda i, be: (be[i], 0, 0)),
            pl.BlockSpec((1, H, D), lambda i, be: (be[i], 0, 0)),
            pl.BlockSpec((1, 1, D), lambda i, be: (be[i], 0, 0)),
        ],
        out_specs=pl.BlockSpec((BLK, D), lambda i, be: (i, 0)),
    )
    return pl.pallas_call(
        _expert_kernel,
        grid_spec=grid_spec,
        out_shape=jax.ShapeDtypeStruct((R_MAX, D), _F32),
    )(be, xg, ew1, eb1.reshape(E, 1, H), ew2, eb2.reshape(E, 1, D))


@functools.cache
def _sc_collect_kernel():
    mesh = plsc.VectorSubcoreMesh(core_axis_name="c", subcore_axis_name="s")

    @functools.partial(
        pl.kernel,
        out_type=jax.ShapeDtypeStruct((K, N, D), _F32),
        mesh=mesh,
        scratch_types=[
            pltpu.VMEM((CCH, D), _F32),
            pltpu.VMEM((CCH, D), _F32),
            pltpu.VMEM((NCC, CCH), _I32),
            pltpu.SemaphoreType.DMA,
            pltpu.SemaphoreType.DMA,
            pltpu.SemaphoreType.DMA,
            pltpu.SemaphoreType.DMA,
        ],
    )
    def collect(y_hbm, dest_hbm, yg_hbm, bufa, bufb, idx_v, ga, gb, sa, sb):
        wid = lax.axis_index("s") * 2 + lax.axis_index("c")
        slot = wid % 2
        pair = wid // 2
        drow = pair + slot * NTB
        tb = pair * TPC
        for c in range(NCC):
            pltpu.sync_copy(dest_hbm.at[drow, pl.ds(c * CCH, CCH)],
                            idx_v.at[c])
        bufs = (bufa, bufb)
        gsems = (ga, gb)
        ssems = (sa, sb)
        g_cp = [None] * NCC
        s_cp = [None] * NCC
        for c in range(NCC):
            if c >= 2:
                s_cp[c - 2].wait()
            g_cp[c] = pltpu.async_copy(y_hbm.at[idx_v.at[c]], bufs[c % 2],
                                       gsems[c % 2])
            if c >= 1:
                g_cp[c - 1].wait()
                s_cp[c - 1] = pltpu.async_copy(
                    bufs[(c - 1) % 2],
                    yg_hbm.at[slot, pl.ds(tb + (c - 1) * CCH, CCH)],
                    ssems[(c - 1) % 2])
        g_cp[NCC - 1].wait()
        s_cp[NCC - 1] = pltpu.async_copy(
            bufs[(NCC - 1) % 2],
            yg_hbm.at[slot, pl.ds(tb + (NCC - 1) * CCH, CCH)],
            ssems[(NCC - 1) % 2])
        s_cp[NCC - 2].wait()
        s_cp[NCC - 1].wait()

    return collect


def _sc_collect(y, dest2):
    return _sc_collect_kernel()(y, dest2)


def _shared_kernel(x_ref, w1_ref, b1_ref, w2_ref, b2_ref,
                   yg_ref, g0_ref, g1_ref, o_ref):
    h = _gelu(jnp.dot(x_ref[...], w1_ref[...],
                      preferred_element_type=_F32) + b1_ref[...])
    sh = jnp.dot(h, w2_ref[...],
                 preferred_element_type=_F32) + b2_ref[...]
    o_ref[...] = sh + g0_ref[...] * yg_ref[0] + g1_ref[...] * yg_ref[1]


def _shared_call(xf, sw1, sb1r, sw2, sb2r, yg, g0c, g1c):
    return pl.pallas_call(
        _shared_kernel,
        grid=(NTB,),
        in_specs=[
            pl.BlockSpec((TBLK, D), lambda i: (i, 0)),
            pl.BlockSpec((D, H), lambda i: (0, 0)),
            pl.BlockSpec((1, H), lambda i: (0, 0)),
            pl.BlockSpec((H, D), lambda i: (0, 0)),
            pl.BlockSpec((1, D), lambda i: (0, 0)),
            pl.BlockSpec((K, TBLK, D), lambda i: (0, i, 0)),
            pl.BlockSpec((TBLK, 1), lambda i: (i, 0)),
            pl.BlockSpec((TBLK, 1), lambda i: (i, 0)),
        ],
        out_specs=pl.BlockSpec((TBLK, D), lambda i: (i, 0)),
        out_shape=jax.ShapeDtypeStruct((N, D), _F32),
    )(xf, sw1, sb1r, sw2, sb2r, yg, g0c, g1c)


def kernel(x, Wr, br, sW1, sb1, sW2, sb2, eW1, eb1, eW2, eb2):
    b, s, d = x.shape
    xf = x.reshape(N, D)
    g0, g1, dest2, bev = _plan_call(xf, Wr, br.reshape(1, E))
    be = bev[0, :G_MAX]
    xg = _sc_dispatch(xf, dest2)
    y = _expert_call(be, xg, eW1, eb1, eW2, eb2)
    yg = _sc_collect(y, dest2)
    out = _shared_call(xf, sW1, sb1.reshape(1, H),
                       sW2, sb2.reshape(1, D),
                       yg, g0.reshape(N, 1), g1.reshape(N, 1))
    return out.reshape(b, s, d)
